# speedup vs baseline: 1.4919x; 1.4919x over previous
import functools

import jax
import jax.numpy as jnp
from jax import lax
from jax.experimental import pallas as pl
from jax.experimental.pallas import tpu as pltpu
from jax.experimental.pallas import tpu_sc as plsc

VOCAB = 100000
D = 768
BATCH = 4
SEQ = 2048
LANES = 16
VECS = D // LANES

_info = plsc.get_sparse_core_info()
NC = _info.num_cores
NS = _info.num_subcores
NW = NC * NS
S_PER_W = SEQ // NW


def _emb_kernel(x_hbm, tgt_hbm, pos_hbm, out_hbm, idx_v, pos_v, rows_v, gsem, ssem):
    wid = lax.axis_index("s") * NC + lax.axis_index("c")
    s0 = wid * S_PER_W

    pltpu.sync_copy(pos_hbm.at[pl.ds(s0, S_PER_W)], pos_v)

    for b in range(BATCH):
        pltpu.sync_copy(x_hbm.at[b, pl.ds(s0, S_PER_W)], idx_v.at[b])
        pltpu.async_copy(tgt_hbm.at[idx_v.at[b]], rows_v, gsem).wait()

        def body(r, carry):
            for k in range(VECS):
                v = pos_v[r, pl.ds(k * LANES, LANES)]
                plsc.addupdate(rows_v.at[r, pl.ds(k * LANES, LANES)], v)
            return carry

        lax.fori_loop(0, S_PER_W, body, 0)

        pltpu.async_copy(rows_v, out_hbm.at[b, pl.ds(s0, S_PER_W)], ssem).wait()


@jax.jit
def _emb(x, tgt_emb, pos_emb):
    mesh = plsc.VectorSubcoreMesh(core_axis_name="c", subcore_axis_name="s")
    f = functools.partial(
        pl.kernel,
        out_type=jax.ShapeDtypeStruct((BATCH, SEQ, D), jnp.float32),
        mesh=mesh,
        scratch_types=[
            pltpu.VMEM((BATCH, S_PER_W), jnp.int32),
            pltpu.VMEM((S_PER_W, D), jnp.float32),
            pltpu.VMEM((S_PER_W, D), jnp.float32),
            pltpu.SemaphoreType.DMA,
            pltpu.SemaphoreType.DMA,
        ],
    )(_emb_kernel)
    return f(x, tgt_emb, pos_emb)


def kernel(x, tgt_emb, pos_emb):
    return _emb(x.astype(jnp.int32), tgt_emb, pos_emb)
